# baseline (speedup 1.0000x reference)
import jax
import jax.numpy as jnp
from jax.experimental import pallas as pl


def _add_kernel(src_ref, pos_ref, out_ref):
    out_ref[...] = src_ref[...] + pos_ref[...][None]


def kernel(src, pos_embed):
    B, S, D = src.shape
    BB = 8
    return pl.pallas_call(
        _add_kernel,
        grid=(B // BB,),
        in_specs=[
            pl.BlockSpec((BB, S, D), lambda i: (i, 0, 0)),
            pl.BlockSpec((S, D), lambda i: (0, 0)),
        ],
        out_specs=pl.BlockSpec((BB, S, D), lambda i: (i, 0, 0)),
        out_shape=jax.ShapeDtypeStruct((B, S, D), src.dtype),
    )(src, pos_embed)


# speedup vs baseline: 1.0954x; 1.0954x over previous
import jax
import jax.numpy as jnp
from jax.experimental import pallas as pl


def _add_kernel(src_ref, pos_ref, out_ref):
    out_ref[...] = src_ref[...] + pos_ref[...][None]


def kernel(src, pos_embed):
    B, S, D = src.shape
    BB = 16
    return pl.pallas_call(
        _add_kernel,
        grid=(B // BB,),
        in_specs=[
            pl.BlockSpec((BB, S, D), lambda i: (i, 0, 0)),
            pl.BlockSpec((S, D), lambda i: (0, 0)),
        ],
        out_specs=pl.BlockSpec((BB, S, D), lambda i: (i, 0, 0)),
        out_shape=jax.ShapeDtypeStruct((B, S, D), src.dtype),
    )(src, pos_embed)


# speedup vs baseline: 1.1098x; 1.0132x over previous
import jax
import jax.numpy as jnp
from jax.experimental import pallas as pl


def _add_kernel(src_ref, pos_ref, out_ref):
    out_ref[...] = src_ref[...] + pos_ref[...][None]


def kernel(src, pos_embed):
    B, S, D = src.shape
    BB = 32
    return pl.pallas_call(
        _add_kernel,
        grid=(B // BB,),
        in_specs=[
            pl.BlockSpec((BB, S, D), lambda i: (i, 0, 0)),
            pl.BlockSpec((S, D), lambda i: (0, 0)),
        ],
        out_specs=pl.BlockSpec((BB, S, D), lambda i: (i, 0, 0)),
        out_shape=jax.ShapeDtypeStruct((B, S, D), src.dtype),
    )(src, pos_embed)
